# baseline (speedup 1.0000x reference)
import jax
import jax.numpy as jnp
from jax import lax
from jax.experimental import pallas as pl
from jax.experimental.pallas import tpu as pltpu
from jax.experimental.pallas import tpu_sc as plsc

_N = 1600000
_B = 50000
_D = 16
_R = 512
_CH = 128

_NW = 32
_BPW = 1600
_B_PAD = _NW * _BPW
_GCH = 128
_NG = (_BPW + 1 + _GCH - 1) // _GCH
_G = _NG * _GCH
_SME = _BPW + 8
_PTR_LEN = (_NW - 1) * _BPW + _G


def _lrelu(v):
    return jnp.where(v >= 0, v, 0.2 * v)


def _mlp_cumsum_body(xb, hb, w1x, w1h, b1, w2, b2, w3, b3, out, carry):
    @pl.when(pl.program_id(0) == 0)
    def _init():
        carry[...] = jnp.zeros_like(carry)

    z = xb[...] @ w1x[...] + hb[...] @ w1h[...] + b1[...]
    z = _lrelu(z)
    z = _lrelu(z @ w2[...] + b2[...])
    h = z @ w3[...] + b3[...]

    ii = lax.broadcasted_iota(jnp.int32, (_CH, _CH), 0)
    jj = lax.broadcasted_iota(jnp.int32, (_CH, _CH), 1)
    ls = (jj < ii).astype(jnp.float32)

    prev = carry[...]
    for k in range(_R // _CH):
        hc = h[k * _CH:(k + 1) * _CH]
        exc = jax.lax.dot(ls, hc, preferred_element_type=jnp.float32)
        out[k * _CH:(k + 1) * _CH, :] = exc + prev
        prev = prev + exc[_CH - 1:_CH] + hc[_CH - 1:_CH]
    carry[...] = prev


def _seg_diff_body(e_hbm, ptr_hbm, out_hbm, idx_v, gidx_v, out_v,
                   b0, b1, b2, sem0, sem1, sem2):
    bufs = (b0, b1, b2)
    sems = (sem0, sem1, sem2)
    ptr_s = idx_v
    wid = lax.axis_index("s") * 2 + lax.axis_index("c")
    base = wid * _BPW
    pltpu.sync_copy(ptr_hbm.at[pl.ds(base, _G)], idx_v)

    def mk_gidx(j, _):
        v = idx_v[pl.ds(j * 16, 16)]
        gidx_v[pl.ds(j * 16, 16)] = lax.shift_right_logical(v, 3)
        return 0
    lax.fori_loop(0, _G // 16, mk_gidx, 0)

    def issue(c):
        return pltpu.async_copy(
            e_hbm.at[gidx_v.at[pl.ds(c * _GCH, _GCH)]],
            bufs[c % 3], sems[c % 3])

    c0 = issue(0)
    c1 = issue(1)
    pending = [None, None, None]
    pending[0] = c0
    pending[1] = c1
    pending[0].wait()

    def sub16(buf_lo, i_lo, p_lo, buf_hi, i_hi, p_hi):
        lo = buf_lo[i_lo, pl.ds((p_lo & 7) * _D, _D)]
        hi = buf_hi[i_hi, pl.ds((p_hi & 7) * _D, _D)]
        return hi - lo

    for c in range(_NG):
        if c + 2 < _NG:
            pending[(c + 2) % 3] = issue(c + 2)
        if c + 1 < _NG:
            pending[(c + 1) % 3].wait()
        ba = bufs[c % 3]
        nrows = min(_GCH, _BPW - c * _GCH)
        in_chunk = nrows if nrows < _GCH else _GCH - 1

        def row(i, _):
            s = c * _GCH + i
            pv = ptr_s[pl.ds(s, 16)]
            p_lo = pv[0]
            p_hi = pv[1]
            out_v[s >> 3, pl.ds((s & 7) * _D, _D)] = sub16(
                ba, i, p_lo, ba, i + 1, p_hi)
            return 0
        lax.fori_loop(0, in_chunk, row, 0)
        if in_chunk < nrows:
            s = c * _GCH + _GCH - 1
            pv = ptr_s[pl.ds(s, 16)]
            p_lo = pv[0]
            p_hi = pv[1]
            out_v[s >> 3, pl.ds((s & 7) * _D, _D)] = sub16(
                ba, _GCH - 1, p_lo, bufs[(c + 1) % 3], 0, p_hi)

    pltpu.sync_copy(out_v, out_hbm.at[pl.ds(wid * (_BPW // 8), _BPW // 8)])


def kernel(h_node, x, ptr, W1, b1, W2, b2, W3, b3):
    n = h_node.shape[0]
    nf = x.shape[1]
    w1x = W1[:, :nf].T
    w1h = W1[:, nf:].T
    w2 = W2.T
    w3 = W3.T
    b1r = b1.reshape(1, -1)
    b2r = b2.reshape(1, -1)
    b3r = b3.reshape(1, -1)

    e2 = pl.pallas_call(
        _mlp_cumsum_body,
        grid=(n // _R,),
        in_specs=[
            pl.BlockSpec((_R, nf), lambda i: (i, 0)),
            pl.BlockSpec((_R, _D), lambda i: (i, 0)),
            pl.BlockSpec(w1x.shape, lambda i: (0, 0)),
            pl.BlockSpec(w1h.shape, lambda i: (0, 0)),
            pl.BlockSpec(b1r.shape, lambda i: (0, 0)),
            pl.BlockSpec(w2.shape, lambda i: (0, 0)),
            pl.BlockSpec(b2r.shape, lambda i: (0, 0)),
            pl.BlockSpec(w3.shape, lambda i: (0, 0)),
            pl.BlockSpec(b3r.shape, lambda i: (0, 0)),
        ],
        out_specs=pl.BlockSpec((_R, _D), lambda i: (i, 0)),
        out_shape=jax.ShapeDtypeStruct((n, _D), jnp.float32),
        scratch_shapes=[pltpu.VMEM((1, _D), jnp.float32)],
    )(x, h_node, w1x, w1h, b1r, w2, b2r, w3, b3r)
    e2 = e2.reshape(n // 8, 8 * _D)

    ptr32 = jnp.pad(ptr.astype(jnp.int32), (0, _PTR_LEN - (_B + 1)),
                    mode="edge")

    seg = pl.kernel(
        _seg_diff_body,
        out_type=jax.ShapeDtypeStruct((_B_PAD // 8, 8 * _D), jnp.float32),
        mesh=plsc.VectorSubcoreMesh(core_axis_name="c", subcore_axis_name="s"),
        scratch_types=[
            pltpu.VMEM((_G,), jnp.int32),
            pltpu.VMEM((_G,), jnp.int32),
            pltpu.VMEM((_BPW // 8, 8 * _D), jnp.float32),
            pltpu.VMEM((_GCH, 8 * _D), jnp.float32),
            pltpu.VMEM((_GCH, 8 * _D), jnp.float32),
            pltpu.VMEM((_GCH, 8 * _D), jnp.float32),
            pltpu.SemaphoreType.DMA,
            pltpu.SemaphoreType.DMA,
            pltpu.SemaphoreType.DMA,
        ],
    )(e2, ptr32)

    return seg.reshape(_B_PAD, _D)[:_B]


# speedup vs baseline: 2.7792x; 2.7792x over previous
import numpy as np

import jax
import jax.numpy as jnp
from jax import lax
from jax.experimental import pallas as pl
from jax.experimental.pallas import tpu as pltpu
from jax.experimental.pallas import tpu_sc as plsc

_N = 1600000
_B = 50000
_D = 16
_R8 = 800
_C = 160

_NW = 32
_BPW = 1600
_B_PAD = _NW * _BPW
_GCH = 128
_NG = (_BPW + 1 + _GCH - 1) // _GCH
_G = _NG * _GCH
_PTR_LEN = (_NW - 1) * _BPW + _G


def _lrelu(v):
    return jnp.where(v >= 0, v, 0.2 * v)


def _mlp_cumsum_body(x8, h128, w1xbd, w1hbd, b1t, w2bd, b2t, w3it, cvec,
                     lsc, out, carry):
    @pl.when(pl.program_id(0) == 0)
    def _init():
        carry[...] = jnp.zeros_like(carry)

    z1 = _lrelu(x8[...] @ w1xbd[...] + h128[...] @ w1hbd[...] + b1t[...])
    z2 = _lrelu(z1 @ w2bd[...] + b2t[...])
    it = z2 @ w3it[...] + cvec[...]

    prev = carry[...]
    for c in range(_R8 // _C):
        tb = it[c * _C:(c + 1) * _C, 128:]
        ic = jax.lax.dot(lsc[...], tb, preferred_element_type=jnp.float32)
        out[c * _C:(c + 1) * _C, :] = (
            it[c * _C:(c + 1) * _C, :128] + ic + prev)
        prev = prev + ic[_C - 1:_C] + tb[_C - 1:_C]
    carry[...] = prev


def _seg_diff_body(e_hbm, ptr_hbm, out_hbm, idx_v, gidx_v, out_v,
                   b0, b1, b2, sem0, sem1, sem2):
    bufs = (b0, b1, b2)
    sems = (sem0, sem1, sem2)
    ptr_s = idx_v
    wid = lax.axis_index("s") * 2 + lax.axis_index("c")
    base = wid * _BPW
    pltpu.sync_copy(ptr_hbm.at[pl.ds(base, _G)], idx_v)

    def mk_gidx(j, _):
        v = idx_v[pl.ds(j * 16, 16)]
        gidx_v[pl.ds(j * 16, 16)] = lax.shift_right_logical(v, 3)
        return 0
    lax.fori_loop(0, _G // 16, mk_gidx, 0)

    def issue(c):
        return pltpu.async_copy(
            e_hbm.at[gidx_v.at[pl.ds(c * _GCH, _GCH)]],
            bufs[c % 3], sems[c % 3])

    pending = [None, None, None]
    pending[0] = issue(0)
    pending[1] = issue(1)
    pending[0].wait()

    def sub16(buf_lo, i_lo, p_lo, buf_hi, i_hi, p_hi):
        lo = buf_lo[i_lo, pl.ds((p_lo & 7) * _D, _D)]
        hi = buf_hi[i_hi, pl.ds((p_hi & 7) * _D, _D)]
        return hi - lo

    for c in range(_NG):
        if c + 2 < _NG:
            pending[(c + 2) % 3] = issue(c + 2)
        if c + 1 < _NG:
            pending[(c + 1) % 3].wait()
        ba = bufs[c % 3]
        nrows = min(_GCH, _BPW - c * _GCH)
        in_chunk = nrows if nrows < _GCH else _GCH - 1

        def row(i, _):
            s = c * _GCH + i
            pv = ptr_s[pl.ds(s, 16)]
            p_lo = pv[0]
            p_hi = pv[1]
            out_v[s >> 3, pl.ds((s & 7) * _D, _D)] = sub16(
                ba, i, p_lo, ba, i + 1, p_hi)
            return 0
        lax.fori_loop(0, in_chunk, row, 0)
        if in_chunk < nrows:
            s = c * _GCH + _GCH - 1
            pv = ptr_s[pl.ds(s, 16)]
            p_lo = pv[0]
            p_hi = pv[1]
            out_v[s >> 3, pl.ds((s & 7) * _D, _D)] = sub16(
                ba, _GCH - 1, p_lo, bufs[(c + 1) % 3], 0, p_hi)

    pltpu.sync_copy(out_v, out_hbm.at[pl.ds(wid * (_BPW // 8), _BPW // 8)])


def _block_diag8(w):
    k, m = w.shape
    out = jnp.zeros((8 * k, 8 * m), dtype=w.dtype)
    for s in range(8):
        out = out.at[s * k:(s + 1) * k, s * m:(s + 1) * m].set(w)
    return out


def kernel(h_node, x, ptr, W1, b1, W2, b2, W3, b3):
    n = h_node.shape[0]
    nf = x.shape[1]
    n8 = n // 8

    w1xbd = _block_diag8(W1[:, :nf].T)
    w1hbd = _block_diag8(W1[:, nf:].T)
    w2bd = _block_diag8(W2.T)
    w3bd = _block_diag8(W3.T)
    b1t = jnp.tile(b1, 8).reshape(1, -1)
    b2t = jnp.tile(b2, 8).reshape(1, -1)
    b3t = jnp.tile(b3, 8).reshape(1, -1)

    l_intra = jnp.asarray(np.kron(np.triu(np.ones((8, 8)), 1),
                                  np.eye(_D)), dtype=jnp.float32)
    t_bcast = jnp.asarray(np.kron(np.ones((8, 8)), np.eye(_D)),
                          dtype=jnp.float32)
    w3it = jnp.concatenate([w3bd @ l_intra, w3bd @ t_bcast], axis=1)
    cvec = jnp.concatenate([b3t @ l_intra, b3t @ t_bcast], axis=1)
    lsc = jnp.asarray(np.tril(np.ones((_C, _C)), -1), dtype=jnp.float32)

    x8 = x.reshape(n8, 8 * nf)
    h128 = h_node.reshape(n8, 8 * _D)

    e2 = pl.pallas_call(
        _mlp_cumsum_body,
        grid=(n8 // _R8,),
        in_specs=[
            pl.BlockSpec((_R8, 8 * nf), lambda i: (i, 0)),
            pl.BlockSpec((_R8, 8 * _D), lambda i: (i, 0)),
            pl.BlockSpec(w1xbd.shape, lambda i: (0, 0)),
            pl.BlockSpec(w1hbd.shape, lambda i: (0, 0)),
            pl.BlockSpec(b1t.shape, lambda i: (0, 0)),
            pl.BlockSpec(w2bd.shape, lambda i: (0, 0)),
            pl.BlockSpec(b2t.shape, lambda i: (0, 0)),
            pl.BlockSpec(w3it.shape, lambda i: (0, 0)),
            pl.BlockSpec(cvec.shape, lambda i: (0, 0)),
            pl.BlockSpec(lsc.shape, lambda i: (0, 0)),
        ],
        out_specs=pl.BlockSpec((_R8, 8 * _D), lambda i: (i, 0)),
        out_shape=jax.ShapeDtypeStruct((n8, 8 * _D), jnp.float32),
        scratch_shapes=[pltpu.VMEM((1, 8 * _D), jnp.float32)],
    )(x8, h128, w1xbd, w1hbd, b1t, w2bd, b2t, w3it, cvec, lsc)

    ptr32 = jnp.pad(ptr.astype(jnp.int32), (0, _PTR_LEN - (_B + 1)),
                    mode="edge")

    seg = pl.kernel(
        _seg_diff_body,
        out_type=jax.ShapeDtypeStruct((_B_PAD // 8, 8 * _D), jnp.float32),
        mesh=plsc.VectorSubcoreMesh(core_axis_name="c", subcore_axis_name="s"),
        scratch_types=[
            pltpu.VMEM((_G,), jnp.int32),
            pltpu.VMEM((_G,), jnp.int32),
            pltpu.VMEM((_BPW // 8, 8 * _D), jnp.float32),
            pltpu.VMEM((_GCH, 8 * _D), jnp.float32),
            pltpu.VMEM((_GCH, 8 * _D), jnp.float32),
            pltpu.VMEM((_GCH, 8 * _D), jnp.float32),
            pltpu.SemaphoreType.DMA,
            pltpu.SemaphoreType.DMA,
            pltpu.SemaphoreType.DMA,
        ],
    )(e2, ptr32)

    return seg.reshape(_B_PAD, _D)[:_B]
